# baseline (speedup 1.0000x reference)
import functools

import jax
import jax.numpy as jnp
from jax import lax
from jax.experimental import pallas as pl
from jax.experimental.pallas import tpu as pltpu
from jax.experimental.pallas import tpu_sc as plsc

N_NODES = 10000
N_EDGES = 320000
D = 128
OUT_SIZE = 3

CHUNK = 128
NCHUNK = N_EDGES // CHUNK
NCORES = 2
NSUB = 16
NW = NCORES * NSUB
NCH_PAD = 2560
SUB_ROWS = 624
TAIL_ROWS = N_NODES - NSUB * SUB_ROWS

BE = 512
BN = 1000

_F32 = jnp.float32


def _dot(a, b):
    return jnp.dot(a, b, preferred_element_type=_F32)


def _dotb(a, b):
    return jnp.dot(a.astype(jnp.bfloat16), b.astype(jnp.bfloat16),
                   preferred_element_type=_F32)


def _ln(z, g, b):
    m = jnp.mean(z, axis=-1, keepdims=True)
    v = jnp.mean((z - m) ** 2, axis=-1, keepdims=True)
    return (z - m) * lax.rsqrt(v + 1e-5) * g + b


def _full(a):
    return pl.BlockSpec(a.shape, lambda i: (0,) * a.ndim)


def _enc_body(x_ref, w1, b1, w2, b2, w3, b3, g, bt, o_ref):
    z = jnp.maximum(_dot(x_ref[...], w1[...]) + b1[...], 0.0)
    z = jnp.maximum(_dot(z, w2[...]) + b2[...], 0.0)
    z = _dot(z, w3[...]) + b3[...]
    o_ref[...] = _ln(z, g[...], bt[...])


def _enc_call(h, p, block, rows=None, off_blocks=0):
    n, din = h.shape
    rows = n if rows is None else rows
    w1, w2, w3 = p["W"]
    b1, b2, b3 = [b.reshape(1, D) for b in p["b"]]
    g = p["ln_g"].reshape(1, D)
    bt = p["ln_b"].reshape(1, D)
    return pl.pallas_call(
        _enc_body,
        grid=(rows // block,),
        in_specs=[pl.BlockSpec((block, din), lambda i: (i + off_blocks, 0)),
                  _full(w1), _full(b1), _full(w2), _full(b2),
                  _full(w3), _full(b3), _full(g), _full(bt)],
        out_specs=pl.BlockSpec((block, D), lambda i: (i, 0)),
        out_shape=jax.ShapeDtypeStruct((rows, D), _F32),
    )(h, w1, b1, w2, b2, w3, b3, g, bt)


def _proj_body(n_ref, ws, wd, ps_ref, pd_ref):
    n = n_ref[...]
    ps_ref[...] = _dot(n, ws[...])
    pd_ref[...] = _dot(n, wd[...])


def _proj_call(node, ws, wd):
    return pl.pallas_call(
        _proj_body,
        grid=(N_NODES // BN,),
        in_specs=[pl.BlockSpec((BN, D), lambda i: (i, 0)),
                  _full(ws), _full(wd)],
        out_specs=[pl.BlockSpec((BN, D), lambda i: (i, 0))] * 2,
        out_shape=[jax.ShapeDtypeStruct((N_NODES, D), _F32)] * 2,
    )(node, ws, wd)


def _edge_body(e_ref, gs_ref, gd_ref, w1, b1, w2, b2, w3, b3, g, bt, o_ref):
    e = e_ref[...]
    z = _dotb(e, w1[...]) + gs_ref[...] + gd_ref[...] + b1[...]
    z = jnp.maximum(z, 0.0)
    z = jnp.maximum(_dotb(z, w2[...]) + b2[...], 0.0)
    z = _dotb(z, w3[...]) + b3[...]
    o_ref[...] = e + _ln(z, g[...], bt[...])


def _edge_call(edge, gs, gd, w1e, b1, w2, b2, w3, b3, g, bt):
    rows = edge.shape[0]
    bs = pl.BlockSpec((BE, D), lambda i: (i, 0))
    return pl.pallas_call(
        _edge_body,
        grid=(rows // BE,),
        in_specs=[bs, bs, bs, _full(w1e), _full(b1), _full(w2), _full(b2),
                  _full(w3), _full(b3), _full(g), _full(bt)],
        out_specs=bs,
        out_shape=jax.ShapeDtypeStruct((rows, D), _F32),
        input_output_aliases={0: 0},
    )(edge, gs, gd, w1e, b1, w2, b2, w3, b3, g, bt)


def _node_body(n_ref, aa_ref, ab_ref, ac_ref, ad_ref,
               wn, wa, b1, w2, b2, w3, b3, g, bt, o_ref):
    n = n_ref[...]
    agg = aa_ref[...] + ab_ref[...] + ac_ref[...] + ad_ref[...]
    z = _dot(n, wn[...]) + _dot(agg, wa[...]) + b1[...]
    z = jnp.maximum(z, 0.0)
    z = jnp.maximum(_dot(z, w2[...]) + b2[...], 0.0)
    z = _dot(z, w3[...]) + b3[...]
    o_ref[...] = n + _ln(z, g[...], bt[...])


def _node_call(node, agg_lo, agg_hi, wn, wa, b1, w2, b2, w3, b3, g, bt):
    bs = pl.BlockSpec((BN, D), lambda i: (i, 0))
    bs_b = pl.BlockSpec((BN, D), lambda i: (i + N_NODES // BN, 0))
    return pl.pallas_call(
        _node_body,
        grid=(N_NODES // BN,),
        in_specs=[bs, bs, bs_b, bs, bs_b,
                  _full(wn), _full(wa), _full(b1), _full(w2),
                  _full(b2), _full(w3), _full(b3), _full(g), _full(bt)],
        out_specs=bs,
        out_shape=jax.ShapeDtypeStruct((N_NODES, D), _F32),
    )(node, agg_lo, agg_lo, agg_hi, agg_hi,
      wn, wa, b1, w2, b2, w3, b3, g, bt)


def _dec_body(n_ref, w1, b1, w2, b2, w3, b3, o_ref):
    z = jnp.maximum(_dot(n_ref[...], w1[...]) + b1[...], 0.0)
    z = jnp.maximum(_dot(z, w2[...]) + b2[...], 0.0)
    o_ref[...] = _dot(z, w3[...]) + b3[...]


def _dec_call(node, p):
    w1, w2, w3 = p["W"]
    b1, b2 = [b.reshape(1, D) for b in p["b"][:2]]
    w3p = jnp.zeros((D, D), _F32).at[:, :OUT_SIZE].set(w3)
    b3p = jnp.zeros((1, D), _F32).at[:, :OUT_SIZE].set(p["b"][2].reshape(1, -1))
    out = pl.pallas_call(
        _dec_body,
        grid=(N_NODES // BN,),
        in_specs=[pl.BlockSpec((BN, D), lambda i: (i, 0)), _full(w1), _full(b1),
                  _full(w2), _full(b2), _full(w3p), _full(b3p)],
        out_specs=pl.BlockSpec((BN, D), lambda i: (i, 0)),
        out_shape=jax.ShapeDtypeStruct((N_NODES, D), _F32),
    )(node, w1, b1, w2, b2, w3p, b3p)
    return out[:, :OUT_SIZE]


H0_CH = 1280
H1_CH = NCHUNK - H0_CH
WCHH = 40
NB = 2
NB_S = 2
NGRP_S = -(-WCHH // NB_S)

def _sc_mesh():
    return plsc.VectorSubcoreMesh(core_axis_name="c", subcore_axis_name="s",
                                  num_cores=NCORES, num_subcores=NSUB)


def _make_gather2(c_start, nch):
    wpt = NSUB
    wch2 = ((-(-nch // wpt)) + 7) // 8 * 8

    def body(ps_hbm, pd_hbm, si2_hbm, di2_hbm, gs_hbm, gd_hbm,
             idx_v, bufs, tab_sh, gsems, osems):
        cid = lax.axis_index("c")
        sid = lax.axis_index("s")
        c0 = sid * wch2
        nv = jnp.minimum(wch2, jnp.maximum(nch - c0, 0))
        ngrp = -(-wch2 // NB)

        @pl.when(cid == 0)
        def _():
            pltpu.sync_copy(ps_hbm.at[pl.ds(sid * SUB_ROWS, SUB_ROWS)],
                            tab_sh.at[pl.ds(sid * SUB_ROWS, SUB_ROWS)])
            pltpu.sync_copy(si2_hbm.at[pl.ds(c_start + c0, wch2)], idx_v)

        @pl.when(cid == 1)
        def _():
            pltpu.sync_copy(pd_hbm.at[pl.ds(sid * SUB_ROWS, SUB_ROWS)],
                            tab_sh.at[pl.ds(sid * SUB_ROWS, SUB_ROWS)])
            pltpu.sync_copy(di2_hbm.at[pl.ds(c_start + c0, wch2)], idx_v)

        @pl.when((sid == 0) & (cid == 0))
        def _():
            pltpu.sync_copy(ps_hbm.at[pl.ds(NSUB * SUB_ROWS, TAIL_ROWS)],
                            tab_sh.at[pl.ds(NSUB * SUB_ROWS, TAIL_ROWS)])

        @pl.when((sid == 0) & (cid == 1))
        def _():
            pltpu.sync_copy(pd_hbm.at[pl.ds(NSUB * SUB_ROWS, TAIL_ROWS)],
                            tab_sh.at[pl.ds(NSUB * SUB_ROWS, TAIL_ROWS)])

        plsc.subcore_barrier()

        def ring(out_hbm):
            def out_cp(it, b):
                base = (c0 + it) * CHUNK
                return pltpu.make_async_copy(
                    bufs[b], out_hbm.at[pl.ds(base, CHUNK)], osems[b])

            def group(gi, carry):
                for b in range(NB):
                    it = (gi - 1) * NB + b

                    @pl.when((gi > 0) & (it < nv))
                    def _(it=it, b=b):
                        out_cp(it, b).wait()

                for b in range(NB):
                    it = gi * NB + b

                    @pl.when(it < nv)
                    def _(it=it, b=b):
                        pltpu.async_copy(tab_sh.at[idx_v.at[it]], bufs[b],
                                         gsems[b])

                for b in range(NB):
                    it = gi * NB + b

                    @pl.when(it < nv)
                    def _(it=it, b=b):
                        pltpu.make_async_copy(tab_sh.at[idx_v.at[it]],
                                              bufs[b], gsems[b]).wait()
                        out_cp(it, b).start()

                return carry

            lax.fori_loop(0, ngrp, group, None)
            for b in range(NB):
                it = (ngrp - 1) * NB + b

                @pl.when(it < nv)
                def _(it=it, b=b):
                    out_cp(it, b).wait()

        @pl.when(cid == 0)
        def _():
            ring(gs_hbm)

        @pl.when(cid == 1)
        def _():
            ring(gd_hbm)

    return pl.kernel(
        body,
        out_type=[jax.ShapeDtypeStruct((nch * CHUNK, D), _F32)] * 2,
        mesh=_sc_mesh(),
        scratch_types=[
            pltpu.VMEM((wch2, CHUNK), jnp.int32),
            [pltpu.VMEM((CHUNK, D), _F32)] * NB,
            pltpu.VMEM_SHARED((N_NODES, D), _F32),
            [pltpu.SemaphoreType.DMA] * NB,
            [pltpu.SemaphoreType.DMA] * NB,
        ],
    )


def _make_segsum(c_start, nch):
    def body(e_hbm, di2_hbm, z_hbm, out_hbm, didx_v, ebufs, agg_sh,
             lsems, ssems):
        cid = lax.axis_index("c")
        sid = lax.axis_index("s")
        wid = sid * NCORES + cid
        c0 = wid * WCHH
        nv = jnp.minimum(WCHH, jnp.maximum(nch - c0, 0))

        pltpu.sync_copy(z_hbm.at[pl.ds(sid * SUB_ROWS, SUB_ROWS)],
                        agg_sh.at[pl.ds(sid * SUB_ROWS, SUB_ROWS)])

        @pl.when(sid == 0)
        def _():
            pltpu.sync_copy(z_hbm.at[pl.ds(NSUB * SUB_ROWS, TAIL_ROWS)],
                            agg_sh.at[pl.ds(NSUB * SUB_ROWS, TAIL_ROWS)])

        pltpu.sync_copy(di2_hbm.at[pl.ds(c_start + c0, WCHH)], didx_v)
        plsc.subcore_barrier()

        def scat(it, b):
            return pltpu.make_async_copy(ebufs[b], agg_sh.at[didx_v.at[it]],
                                         ssems[b])

        def group(gi, carry):
            for b in range(NB_S):
                it = (gi - 1) * NB_S + b

                @pl.when((gi > 0) & (it < nv))
                def _(it=it, b=b):
                    scat(it, b).wait()

            for b in range(NB_S):
                it = gi * NB_S + b

                @pl.when(it < nv)
                def _(it=it, b=b):
                    pltpu.async_copy(
                        e_hbm.at[pl.ds((c0 + it) * CHUNK, CHUNK)],
                        ebufs[b], lsems[b])

            for b in range(NB_S):
                it = gi * NB_S + b

                @pl.when(it < nv)
                def _(it=it, b=b):
                    pltpu.make_async_copy(
                        e_hbm.at[pl.ds((c0 + it) * CHUNK, CHUNK)],
                        ebufs[b], lsems[b]).wait()
                    scat(it, b).start(add=True)

            return carry

        lax.fori_loop(0, NGRP_S, group, None)
        for b in range(NB_S):
            it = (NGRP_S - 1) * NB_S + b

            @pl.when(it < nv)
            def _(it=it, b=b):
                scat(it, b).wait()

        plsc.subcore_barrier()
        pltpu.sync_copy(
            agg_sh.at[pl.ds(sid * SUB_ROWS, SUB_ROWS)],
            out_hbm.at[pl.ds(cid * N_NODES + sid * SUB_ROWS, SUB_ROWS)])

        @pl.when(sid == 0)
        def _():
            pltpu.sync_copy(
                agg_sh.at[pl.ds(NSUB * SUB_ROWS, TAIL_ROWS)],
                out_hbm.at[pl.ds(cid * N_NODES + NSUB * SUB_ROWS, TAIL_ROWS)])

    return pl.kernel(
        body,
        out_type=jax.ShapeDtypeStruct((NCORES * N_NODES, D), _F32),
        mesh=_sc_mesh(),
        scratch_types=[
            pltpu.VMEM((WCHH, CHUNK), jnp.int32),
            [pltpu.VMEM((CHUNK, D), _F32)] * NB_S,
            pltpu.VMEM_SHARED((N_NODES, D), _F32),
            [pltpu.SemaphoreType.DMA] * NB_S,
            [pltpu.SemaphoreType.DMA] * NB_S,
        ],
    )


def _stack_proc(proc):
    def st(f):
        return jnp.stack([f(b) for b in proc])

    w1 = st(lambda b: b["edge"]["W"][0])
    return (
        w1[:, :D, :], w1[:, D:2 * D, :], w1[:, 2 * D:, :],
        st(lambda b: b["edge"]["b"][0].reshape(1, D)),
        st(lambda b: b["edge"]["W"][1]),
        st(lambda b: b["edge"]["b"][1].reshape(1, D)),
        st(lambda b: b["edge"]["W"][2]),
        st(lambda b: b["edge"]["b"][2].reshape(1, D)),
        st(lambda b: b["edge"]["ln_g"].reshape(1, D)),
        st(lambda b: b["edge"]["ln_b"].reshape(1, D)),
        st(lambda b: b["node"]["W"][0][:D, :]),
        st(lambda b: b["node"]["W"][0][D:, :]),
        st(lambda b: b["node"]["b"][0].reshape(1, D)),
        st(lambda b: b["node"]["W"][1]),
        st(lambda b: b["node"]["b"][1].reshape(1, D)),
        st(lambda b: b["node"]["W"][2]),
        st(lambda b: b["node"]["b"][2].reshape(1, D)),
        st(lambda b: b["node"]["ln_g"].reshape(1, D)),
        st(lambda b: b["node"]["ln_b"].reshape(1, D)),
    )


def kernel(x, edge_attr, params, edge_index):
    pad = ((0, NCH_PAD - NCHUNK), (0, 0))
    si2 = jnp.pad(edge_index[0].astype(jnp.int32).reshape(NCHUNK, CHUNK), pad)
    di2 = jnp.pad(edge_index[1].astype(jnp.int32).reshape(NCHUNK, CHUNK), pad)

    rows_lo = H0_CH * CHUNK
    node = _enc_call(x, params["enc_node"], BN)
    e_lo = _enc_call(edge_attr, params["enc_edge"], BE, rows=rows_lo)
    e_hi = _enc_call(edge_attr, params["enc_edge"], BE,
                     rows=N_EDGES - rows_lo, off_blocks=rows_lo // BE)
    zeros_nd = jnp.zeros((N_NODES, D), _F32)

    ws = _stack_proc(params["proc"])
    gather_lo = _make_gather2(0, H0_CH)
    gather_hi = _make_gather2(H0_CH, H1_CH)
    segsum_lo = _make_segsum(0, H0_CH)
    segsum_hi = _make_segsum(H0_CH, H1_CH)

    def step(carry, w):
        node, e_lo, e_hi = carry
        (w1e, w1s, w1d, eb1, ew2, eb2, ew3, eb3, eg, ebt,
         wn, wa, nb1, nw2, nb2, nw3, nb3, ng, nbt) = w
        ps, pd = _proj_call(node, w1s, w1d)
        gs_lo, gd_lo = gather_lo(ps, pd, si2, di2)
        gs_hi, gd_hi = gather_hi(ps, pd, si2, di2)
        e_lo = _edge_call(e_lo, gs_lo, gd_lo,
                          w1e, eb1, ew2, eb2, ew3, eb3, eg, ebt)
        a_lo = segsum_lo(e_lo, di2, zeros_nd)
        e_hi = _edge_call(e_hi, gs_hi, gd_hi,
                          w1e, eb1, ew2, eb2, ew3, eb3, eg, ebt)
        a_hi = segsum_hi(e_hi, di2, zeros_nd)
        node = _node_call(node, a_lo, a_hi,
                          wn, wa, nb1, nw2, nb2, nw3, nb3, ng, nbt)
        return (node, e_lo, e_hi), None

    (node, e_lo, e_hi), _ = lax.scan(step, (node, e_lo, e_hi), ws)
    return _dec_call(node, params["dec"])


# speedup vs baseline: 1.0058x; 1.0058x over previous
import functools

import jax
import jax.numpy as jnp
from jax import lax
from jax.experimental import pallas as pl
from jax.experimental.pallas import tpu as pltpu
from jax.experimental.pallas import tpu_sc as plsc

N_NODES = 10000
N_EDGES = 320000
D = 128
OUT_SIZE = 3

CHUNK = 128
NCHUNK = N_EDGES // CHUNK
NCORES = 2
NSUB = 16
NW = NCORES * NSUB
NCH_PAD = 2560
SUB_ROWS = 624
TAIL_ROWS = N_NODES - NSUB * SUB_ROWS

BE = 512
BN = 1000

_F32 = jnp.float32


def _dot(a, b):
    return jnp.dot(a, b, preferred_element_type=_F32)


def _ln(z, g, b):
    m = jnp.mean(z, axis=-1, keepdims=True)
    v = jnp.mean((z - m) ** 2, axis=-1, keepdims=True)
    return (z - m) * lax.rsqrt(v + 1e-5) * g + b


def _full(a):
    return pl.BlockSpec(a.shape, lambda i: (0,) * a.ndim)


def _enc_body(x_ref, w1, b1, w2, b2, w3, b3, g, bt, o_ref):
    z = jnp.maximum(_dot(x_ref[...], w1[...]) + b1[...], 0.0)
    z = jnp.maximum(_dot(z, w2[...]) + b2[...], 0.0)
    z = _dot(z, w3[...]) + b3[...]
    o_ref[...] = _ln(z, g[...], bt[...])


def _enc_call(h, p, block, rows=None, off_blocks=0):
    n, din = h.shape
    rows = n if rows is None else rows
    w1, w2, w3 = p["W"]
    b1, b2, b3 = [b.reshape(1, D) for b in p["b"]]
    g = p["ln_g"].reshape(1, D)
    bt = p["ln_b"].reshape(1, D)
    return pl.pallas_call(
        _enc_body,
        grid=(rows // block,),
        in_specs=[pl.BlockSpec((block, din), lambda i: (i + off_blocks, 0)),
                  _full(w1), _full(b1), _full(w2), _full(b2),
                  _full(w3), _full(b3), _full(g), _full(bt)],
        out_specs=pl.BlockSpec((block, D), lambda i: (i, 0)),
        out_shape=jax.ShapeDtypeStruct((rows, D), _F32),
    )(h, w1, b1, w2, b2, w3, b3, g, bt)


def _proj_body(n_ref, ws, wd, ps_ref, pd_ref):
    n = n_ref[...]
    ps_ref[...] = _dot(n, ws[...])
    pd_ref[...] = _dot(n, wd[...])


def _proj_call(node, ws, wd):
    return pl.pallas_call(
        _proj_body,
        grid=(N_NODES // BN,),
        in_specs=[pl.BlockSpec((BN, D), lambda i: (i, 0)),
                  _full(ws), _full(wd)],
        out_specs=[pl.BlockSpec((BN, D), lambda i: (i, 0))] * 2,
        out_shape=[jax.ShapeDtypeStruct((N_NODES, D), _F32)] * 2,
    )(node, ws, wd)


def _edge_body(e_ref, gs_ref, gd_ref, w1, b1, w2, b2, w3, b3, g, bt, o_ref):
    e = e_ref[...]
    z = _dot(e, w1[...]) + gs_ref[...] + gd_ref[...] + b1[...]
    z = jnp.maximum(z, 0.0)
    z = jnp.maximum(_dot(z, w2[...]) + b2[...], 0.0)
    z = _dot(z, w3[...]) + b3[...]
    o_ref[...] = e + _ln(z, g[...], bt[...])


def _edge_call(edge, gs, gd, w1e, b1, w2, b2, w3, b3, g, bt):
    rows = edge.shape[0]
    bs = pl.BlockSpec((BE, D), lambda i: (i, 0))
    return pl.pallas_call(
        _edge_body,
        grid=(rows // BE,),
        in_specs=[bs, bs, bs, _full(w1e), _full(b1), _full(w2), _full(b2),
                  _full(w3), _full(b3), _full(g), _full(bt)],
        out_specs=bs,
        out_shape=jax.ShapeDtypeStruct((rows, D), _F32),
        input_output_aliases={0: 0},
    )(edge, gs, gd, w1e, b1, w2, b2, w3, b3, g, bt)


def _node_body(n_ref, aa_ref, ab_ref, ac_ref, ad_ref,
               wn, wa, b1, w2, b2, w3, b3, g, bt, o_ref):
    n = n_ref[...]
    agg = aa_ref[...] + ab_ref[...] + ac_ref[...] + ad_ref[...]
    z = _dot(n, wn[...]) + _dot(agg, wa[...]) + b1[...]
    z = jnp.maximum(z, 0.0)
    z = jnp.maximum(_dot(z, w2[...]) + b2[...], 0.0)
    z = _dot(z, w3[...]) + b3[...]
    o_ref[...] = n + _ln(z, g[...], bt[...])


def _node_call(node, agg_lo, agg_hi, wn, wa, b1, w2, b2, w3, b3, g, bt):
    bs = pl.BlockSpec((BN, D), lambda i: (i, 0))
    bs_b = pl.BlockSpec((BN, D), lambda i: (i + N_NODES // BN, 0))
    return pl.pallas_call(
        _node_body,
        grid=(N_NODES // BN,),
        in_specs=[bs, bs, bs_b, bs, bs_b,
                  _full(wn), _full(wa), _full(b1), _full(w2),
                  _full(b2), _full(w3), _full(b3), _full(g), _full(bt)],
        out_specs=bs,
        out_shape=jax.ShapeDtypeStruct((N_NODES, D), _F32),
    )(node, agg_lo, agg_lo, agg_hi, agg_hi,
      wn, wa, b1, w2, b2, w3, b3, g, bt)


def _dec_body(n_ref, w1, b1, w2, b2, w3, b3, o_ref):
    z = jnp.maximum(_dot(n_ref[...], w1[...]) + b1[...], 0.0)
    z = jnp.maximum(_dot(z, w2[...]) + b2[...], 0.0)
    o_ref[...] = _dot(z, w3[...]) + b3[...]


def _dec_call(node, p):
    w1, w2, w3 = p["W"]
    b1, b2 = [b.reshape(1, D) for b in p["b"][:2]]
    w3p = jnp.zeros((D, D), _F32).at[:, :OUT_SIZE].set(w3)
    b3p = jnp.zeros((1, D), _F32).at[:, :OUT_SIZE].set(p["b"][2].reshape(1, -1))
    out = pl.pallas_call(
        _dec_body,
        grid=(N_NODES // BN,),
        in_specs=[pl.BlockSpec((BN, D), lambda i: (i, 0)), _full(w1), _full(b1),
                  _full(w2), _full(b2), _full(w3p), _full(b3p)],
        out_specs=pl.BlockSpec((BN, D), lambda i: (i, 0)),
        out_shape=jax.ShapeDtypeStruct((N_NODES, D), _F32),
    )(node, w1, b1, w2, b2, w3p, b3p)
    return out[:, :OUT_SIZE]


H0_CH = 1280
H1_CH = NCHUNK - H0_CH
WCHH = 40
NB = 2
NB_S = 2
NGRP_S = -(-WCHH // NB_S)

def _sc_mesh():
    return plsc.VectorSubcoreMesh(core_axis_name="c", subcore_axis_name="s",
                                  num_cores=NCORES, num_subcores=NSUB)


def _make_gather2(c_start, nch):
    wpt = NSUB
    wch2 = ((-(-nch // wpt)) + 7) // 8 * 8

    def body(ps_hbm, pd_hbm, si2_hbm, di2_hbm, gs_hbm, gd_hbm,
             idx_v, bufs, tab_sh, gsems, osems):
        cid = lax.axis_index("c")
        sid = lax.axis_index("s")
        c0 = sid * wch2
        nv = jnp.minimum(wch2, jnp.maximum(nch - c0, 0))
        ngrp = -(-wch2 // NB)

        @pl.when(cid == 0)
        def _():
            pltpu.sync_copy(ps_hbm.at[pl.ds(sid * SUB_ROWS, SUB_ROWS)],
                            tab_sh.at[pl.ds(sid * SUB_ROWS, SUB_ROWS)])
            pltpu.sync_copy(si2_hbm.at[pl.ds(c_start + c0, wch2)], idx_v)

        @pl.when(cid == 1)
        def _():
            pltpu.sync_copy(pd_hbm.at[pl.ds(sid * SUB_ROWS, SUB_ROWS)],
                            tab_sh.at[pl.ds(sid * SUB_ROWS, SUB_ROWS)])
            pltpu.sync_copy(di2_hbm.at[pl.ds(c_start + c0, wch2)], idx_v)

        @pl.when((sid == 0) & (cid == 0))
        def _():
            pltpu.sync_copy(ps_hbm.at[pl.ds(NSUB * SUB_ROWS, TAIL_ROWS)],
                            tab_sh.at[pl.ds(NSUB * SUB_ROWS, TAIL_ROWS)])

        @pl.when((sid == 0) & (cid == 1))
        def _():
            pltpu.sync_copy(pd_hbm.at[pl.ds(NSUB * SUB_ROWS, TAIL_ROWS)],
                            tab_sh.at[pl.ds(NSUB * SUB_ROWS, TAIL_ROWS)])

        plsc.subcore_barrier()

        def ring(out_hbm):
            def out_cp(it, b):
                base = (c0 + it) * CHUNK
                return pltpu.make_async_copy(
                    bufs[b], out_hbm.at[pl.ds(base, CHUNK)], osems[b])

            def group(gi, carry):
                for b in range(NB):
                    it = (gi - 1) * NB + b

                    @pl.when((gi > 0) & (it < nv))
                    def _(it=it, b=b):
                        out_cp(it, b).wait()

                for b in range(NB):
                    it = gi * NB + b

                    @pl.when(it < nv)
                    def _(it=it, b=b):
                        pltpu.async_copy(tab_sh.at[idx_v.at[it]], bufs[b],
                                         gsems[b])

                for b in range(NB):
                    it = gi * NB + b

                    @pl.when(it < nv)
                    def _(it=it, b=b):
                        pltpu.make_async_copy(tab_sh.at[idx_v.at[it]],
                                              bufs[b], gsems[b]).wait()
                        out_cp(it, b).start()

                return carry

            lax.fori_loop(0, ngrp, group, None)
            for b in range(NB):
                it = (ngrp - 1) * NB + b

                @pl.when(it < nv)
                def _(it=it, b=b):
                    out_cp(it, b).wait()

        @pl.when(cid == 0)
        def _():
            ring(gs_hbm)

        @pl.when(cid == 1)
        def _():
            ring(gd_hbm)

    return pl.kernel(
        body,
        out_type=[jax.ShapeDtypeStruct((nch * CHUNK, D), _F32)] * 2,
        mesh=_sc_mesh(),
        scratch_types=[
            pltpu.VMEM((wch2, CHUNK), jnp.int32),
            [pltpu.VMEM((CHUNK, D), _F32)] * NB,
            pltpu.VMEM_SHARED((N_NODES, D), _F32),
            [pltpu.SemaphoreType.DMA] * NB,
            [pltpu.SemaphoreType.DMA] * NB,
        ],
    )


def _make_segsum(c_start, nch):
    def body(e_hbm, di2_hbm, z_hbm, out_hbm, didx_v, ebufs, agg_sh,
             lsems, ssems):
        cid = lax.axis_index("c")
        sid = lax.axis_index("s")
        wid = sid * NCORES + cid
        c0 = wid * WCHH
        nv = jnp.minimum(WCHH, jnp.maximum(nch - c0, 0))

        pltpu.sync_copy(z_hbm.at[pl.ds(sid * SUB_ROWS, SUB_ROWS)],
                        agg_sh.at[pl.ds(sid * SUB_ROWS, SUB_ROWS)])

        @pl.when(sid == 0)
        def _():
            pltpu.sync_copy(z_hbm.at[pl.ds(NSUB * SUB_ROWS, TAIL_ROWS)],
                            agg_sh.at[pl.ds(NSUB * SUB_ROWS, TAIL_ROWS)])

        pltpu.sync_copy(di2_hbm.at[pl.ds(c_start + c0, WCHH)], didx_v)
        plsc.subcore_barrier()

        def scat(it, b):
            return pltpu.make_async_copy(ebufs[b], agg_sh.at[didx_v.at[it]],
                                         ssems[b])

        def group(gi, carry):
            for b in range(NB_S):
                it = (gi - 1) * NB_S + b

                @pl.when((gi > 0) & (it < nv))
                def _(it=it, b=b):
                    scat(it, b).wait()

            for b in range(NB_S):
                it = gi * NB_S + b

                @pl.when(it < nv)
                def _(it=it, b=b):
                    pltpu.async_copy(
                        e_hbm.at[pl.ds((c0 + it) * CHUNK, CHUNK)],
                        ebufs[b], lsems[b])

            for b in range(NB_S):
                it = gi * NB_S + b

                @pl.when(it < nv)
                def _(it=it, b=b):
                    pltpu.make_async_copy(
                        e_hbm.at[pl.ds((c0 + it) * CHUNK, CHUNK)],
                        ebufs[b], lsems[b]).wait()
                    scat(it, b).start(add=True)

            return carry

        lax.fori_loop(0, NGRP_S, group, None)
        for b in range(NB_S):
            it = (NGRP_S - 1) * NB_S + b

            @pl.when(it < nv)
            def _(it=it, b=b):
                scat(it, b).wait()

        plsc.subcore_barrier()
        pltpu.sync_copy(
            agg_sh.at[pl.ds(sid * SUB_ROWS, SUB_ROWS)],
            out_hbm.at[pl.ds(cid * N_NODES + sid * SUB_ROWS, SUB_ROWS)])

        @pl.when(sid == 0)
        def _():
            pltpu.sync_copy(
                agg_sh.at[pl.ds(NSUB * SUB_ROWS, TAIL_ROWS)],
                out_hbm.at[pl.ds(cid * N_NODES + NSUB * SUB_ROWS, TAIL_ROWS)])

    return pl.kernel(
        body,
        out_type=jax.ShapeDtypeStruct((NCORES * N_NODES, D), _F32),
        mesh=_sc_mesh(),
        scratch_types=[
            pltpu.VMEM((WCHH, CHUNK), jnp.int32),
            [pltpu.VMEM((CHUNK, D), _F32)] * NB_S,
            pltpu.VMEM_SHARED((N_NODES, D), _F32),
            [pltpu.SemaphoreType.DMA] * NB_S,
            [pltpu.SemaphoreType.DMA] * NB_S,
        ],
    )


def _stack_proc(proc):
    def st(f):
        return jnp.stack([f(b) for b in proc])

    w1 = st(lambda b: b["edge"]["W"][0])
    return (
        w1[:, :D, :], w1[:, D:2 * D, :], w1[:, 2 * D:, :],
        st(lambda b: b["edge"]["b"][0].reshape(1, D)),
        st(lambda b: b["edge"]["W"][1]),
        st(lambda b: b["edge"]["b"][1].reshape(1, D)),
        st(lambda b: b["edge"]["W"][2]),
        st(lambda b: b["edge"]["b"][2].reshape(1, D)),
        st(lambda b: b["edge"]["ln_g"].reshape(1, D)),
        st(lambda b: b["edge"]["ln_b"].reshape(1, D)),
        st(lambda b: b["node"]["W"][0][:D, :]),
        st(lambda b: b["node"]["W"][0][D:, :]),
        st(lambda b: b["node"]["b"][0].reshape(1, D)),
        st(lambda b: b["node"]["W"][1]),
        st(lambda b: b["node"]["b"][1].reshape(1, D)),
        st(lambda b: b["node"]["W"][2]),
        st(lambda b: b["node"]["b"][2].reshape(1, D)),
        st(lambda b: b["node"]["ln_g"].reshape(1, D)),
        st(lambda b: b["node"]["ln_b"].reshape(1, D)),
    )


def kernel(x, edge_attr, params, edge_index):
    pad = ((0, NCH_PAD - NCHUNK), (0, 0))
    si2 = jnp.pad(edge_index[0].astype(jnp.int32).reshape(NCHUNK, CHUNK), pad)
    di2 = jnp.pad(edge_index[1].astype(jnp.int32).reshape(NCHUNK, CHUNK), pad)

    rows_lo = H0_CH * CHUNK
    node = _enc_call(x, params["enc_node"], BN)
    e_lo = _enc_call(edge_attr, params["enc_edge"], BE, rows=rows_lo)
    e_hi = _enc_call(edge_attr, params["enc_edge"], BE,
                     rows=N_EDGES - rows_lo, off_blocks=rows_lo // BE)
    zeros_nd = jnp.zeros((N_NODES, D), _F32)

    ws = _stack_proc(params["proc"])
    gather_lo = _make_gather2(0, H0_CH)
    gather_hi = _make_gather2(H0_CH, H1_CH)
    segsum_lo = _make_segsum(0, H0_CH)
    segsum_hi = _make_segsum(H0_CH, H1_CH)

    def step(carry, w):
        node, e_lo, e_hi = carry
        (w1e, w1s, w1d, eb1, ew2, eb2, ew3, eb3, eg, ebt,
         wn, wa, nb1, nw2, nb2, nw3, nb3, ng, nbt) = w
        ps, pd = _proj_call(node, w1s, w1d)
        gs_lo, gd_lo = gather_lo(ps, pd, si2, di2)
        gs_hi, gd_hi = gather_hi(ps, pd, si2, di2)
        e_lo = _edge_call(e_lo, gs_lo, gd_lo,
                          w1e, eb1, ew2, eb2, ew3, eb3, eg, ebt)
        a_lo = segsum_lo(e_lo, di2, zeros_nd)
        e_hi = _edge_call(e_hi, gs_hi, gd_hi,
                          w1e, eb1, ew2, eb2, ew3, eb3, eg, ebt)
        a_hi = segsum_hi(e_hi, di2, zeros_nd)
        node = _node_call(node, a_lo, a_hi,
                          wn, wa, nb1, nw2, nb2, nw3, nb3, ng, nbt)
        return (node, e_lo, e_hi), None

    (node, e_lo, e_hi), _ = lax.scan(step, (node, e_lo, e_hi), ws)
    return _dec_call(node, params["dec"])
